# baseline (speedup 1.0000x reference)
import jax
import jax.numpy as jnp
from jax import lax
from jax.experimental import pallas as pl
from jax.experimental.pallas import tpu as pltpu
from jax.experimental.pallas import tpu_sc as plsc

_N = 100
_D = 16
_S = 20
_B = 1024
_K = 4
_TILES = _S * _N
_NW = 32
_BASE_T = _TILES // _NW
_EXTRA = _TILES - _BASE_T * _NW


def _sc_kernel(conc_hbm, embx_hbm, out_hbm, conc_v, embx_v, buf_v, sems):
    wid = lax.axis_index("s") * 2 + lax.axis_index("c")
    is_small = (_NW - 1 - wid) // _EXTRA
    t0 = wid * _BASE_T + is_small * wid + (1 - is_small) * _EXTRA
    tend = t0 + _BASE_T + is_small

    pltpu.sync_copy(embx_hbm, embx_v)
    pltpu.sync_copy(conc_hbm.at[t0 // _N], conc_v)

    def compute_tile(t, slot):
        s = t // _N
        n = t - s * _N

        @pl.when((t > t0) & (n == 0))
        def _load_conc():
            pltpu.sync_copy(conc_hbm.at[s], conc_v)

        @pl.when(t >= t0 + 2)
        def _wait_slot():
            pltpu.make_async_copy(
                buf_v.at[slot], out_hbm.at[s, n], sems.at[slot]
            ).wait()

        splats = []
        for d in range(_D):
            splats.append(embx_v[pl.ds((n * _D + d) * _D, _D)])

        def b_body(j, carry):
            off = j * 16
            c0 = conc_v[0, pl.ds(off, 16)]
            c1 = conc_v[1, pl.ds(off, 16)]
            c2 = conc_v[2, pl.ds(off, 16)]
            c3 = conc_v[3, pl.ds(off, 16)]
            miss = (
                jnp.minimum(jnp.abs(c0 - n), 1)
                + jnp.minimum(jnp.abs(c1 - n), 1)
                + jnp.minimum(jnp.abs(c2 - n), 1)
                + jnp.minimum(jnp.abs(c3 - n), 1)
            )
            cnt = (_K - miss).astype(jnp.float32)
            for d in range(_D):
                buf_v[slot, d, pl.ds(off, 16)] = cnt * splats[d]
            return carry

        lax.fori_loop(0, _B // 16, b_body, 0)

        pltpu.make_async_copy(
            buf_v.at[slot], out_hbm.at[s, n], sems.at[slot]
        ).start()

    def pair_body(i, carry):
        t = t0 + 2 * i

        @pl.when(t < tend)
        def _even():
            compute_tile(t, 0)

        @pl.when(t + 1 < tend)
        def _odd():
            compute_tile(t + 1, 1)

        return carry

    lax.fori_loop(0, (_BASE_T + 2) // 2, pair_body, 0)

    for j in range(2):
        pltpu.make_async_copy(buf_v.at[j], out_hbm.at[0, 0], sems.at[j]).wait()


def kernel(concepts, emb_table_skill):
    b, s, k = concepts.shape
    conc_t = jnp.transpose(concepts.astype(jnp.int32), (1, 2, 0))
    embx = jnp.broadcast_to(
        emb_table_skill.reshape(-1)[:, None], ((_N + 1) * _D, _D)
    ).reshape(-1)

    mesh = plsc.VectorSubcoreMesh(core_axis_name="c", subcore_axis_name="s")
    run = pl.kernel(
        _sc_kernel,
        out_type=jax.ShapeDtypeStruct((_S, _N, _D, _B), jnp.float32),
        mesh=mesh,
        scratch_types=[
            pltpu.VMEM((_K, _B), jnp.int32),
            pltpu.VMEM(((_N + 1) * _D * _D,), jnp.float32),
            pltpu.VMEM((2, _D, _B), jnp.float32),
            pltpu.SemaphoreType.DMA((2,)),
        ],
    )
    out_t = run(conc_t, embx)
    return jnp.transpose(out_t, (3, 0, 1, 2))


# speedup vs baseline: 1.5314x; 1.5314x over previous
import jax
import jax.numpy as jnp
from jax import lax
from jax.experimental import pallas as pl
from jax.experimental.pallas import tpu as pltpu
from jax.experimental.pallas import tpu_sc as plsc

_N = 100
_D = 16
_S = 20
_B = 1024
_K = 4
_TILES = _S * _N
_NW = 32
_BASE_T = _TILES // _NW
_EXTRA = _TILES - _BASE_T * _NW


def _sc_kernel(conc_hbm, embx_hbm, out_hbm, conc_v, embx_v, buf_v, sems):
    wid = lax.axis_index("s") * 2 + lax.axis_index("c")
    is_small = (_NW - 1 - wid) // _EXTRA
    t0 = wid * _BASE_T + is_small * wid + (1 - is_small) * _EXTRA
    tend = t0 + _BASE_T + is_small

    pltpu.sync_copy(embx_hbm, embx_v)
    pltpu.sync_copy(conc_hbm.at[t0 // _N], conc_v)

    def compute_tile(t, slot):
        s = t // _N
        n = t - s * _N

        @pl.when((t > t0) & (n == 0))
        def _load_conc():
            pltpu.sync_copy(conc_hbm.at[s], conc_v)

        @pl.when(t >= t0 + 2)
        def _wait_slot():
            pltpu.make_async_copy(
                buf_v.at[slot], out_hbm.at[s, n], sems.at[slot]
            ).wait()

        splats = []
        for d in range(_D):
            splats.append(embx_v[pl.ds((n * _D + d) * _D, _D)])

        @plsc.parallel_loop(0, _B // 16, unroll=4)
        def b_body(j):
            off = j * 16
            c0 = conc_v[0, pl.ds(off, 16)]
            c1 = conc_v[1, pl.ds(off, 16)]
            c2 = conc_v[2, pl.ds(off, 16)]
            c3 = conc_v[3, pl.ds(off, 16)]
            miss = (
                jnp.minimum(jnp.abs(c0 - n), 1)
                + jnp.minimum(jnp.abs(c1 - n), 1)
                + jnp.minimum(jnp.abs(c2 - n), 1)
                + jnp.minimum(jnp.abs(c3 - n), 1)
            )
            cnt = (_K - miss).astype(jnp.float32)
            for d in range(_D):
                buf_v[slot, d, pl.ds(off, 16)] = cnt * splats[d]

        pltpu.make_async_copy(
            buf_v.at[slot], out_hbm.at[s, n], sems.at[slot]
        ).start()

    def pair_body(i, carry):
        t = t0 + 2 * i

        @pl.when(t < tend)
        def _even():
            compute_tile(t, 0)

        @pl.when(t + 1 < tend)
        def _odd():
            compute_tile(t + 1, 1)

        return carry

    lax.fori_loop(0, (_BASE_T + 2) // 2, pair_body, 0)

    for j in range(2):
        pltpu.make_async_copy(buf_v.at[j], out_hbm.at[0, 0], sems.at[j]).wait()


def kernel(concepts, emb_table_skill):
    b, s, k = concepts.shape
    conc_t = jnp.transpose(concepts.astype(jnp.int32), (1, 2, 0))
    embx = jnp.broadcast_to(
        emb_table_skill.reshape(-1)[:, None], ((_N + 1) * _D, _D)
    ).reshape(-1)

    mesh = plsc.VectorSubcoreMesh(core_axis_name="c", subcore_axis_name="s")
    run = pl.kernel(
        _sc_kernel,
        out_type=jax.ShapeDtypeStruct((_S, _N, _D, _B), jnp.float32),
        mesh=mesh,
        scratch_types=[
            pltpu.VMEM((_K, _B), jnp.int32),
            pltpu.VMEM(((_N + 1) * _D * _D,), jnp.float32),
            pltpu.VMEM((2, _D, _B), jnp.float32),
            pltpu.SemaphoreType.DMA((2,)),
        ],
    )
    out_t = run(conc_t, embx)
    return jnp.transpose(out_t, (3, 0, 1, 2))
